# baseline (speedup 1.0000x reference)
import functools

import jax
import jax.numpy as jnp
from jax.experimental import pallas as pl
from jax.experimental.pallas import tpu as pltpu

_NP_RATIO = 3
_THRESHOLD = 0.5
_VAR0, _VAR1 = 0.1, 0.2
_FPB = 8


def _smooth_l1(x):
    ax = jnp.abs(x)
    return jnp.where(ax < 1.0, 0.5 * x * x, ax - 0.5)


def _loss_kernel(tgt_ref, anc_ref, loc_ref, conf_ref, out_l_ref, out_c_ref,
                 ce_ref, np_ref, *, n_frames, n_anchors, n_objs):
    step = pl.program_id(0)
    A = n_anchors
    O = n_objs
    P = _FPB

    @pl.when(step == 0)
    def _init():
        out_l_ref[:, :] = jnp.zeros((1, 1), jnp.float32)
        out_c_ref[:, :] = jnp.zeros((1, 1), jnp.float32)

    anc = anc_ref[:, :]
    cx2, cy2 = anc[0:1, :], anc[1:2, :]
    w2, h2 = anc[2:3, :], anc[3:4, :]
    ax1 = (cx2 - w2 * 0.5)[None]
    ay1 = (cy2 - h2 * 0.5)[None]
    ax2 = (cx2 + w2 * 0.5)[None]
    ay2 = (cy2 + h2 * 0.5)[None]
    area_a = (w2 * h2)[None]

    tgt = tgt_ref[:, :, :]
    tx1, ty1 = tgt[:, :, 0:1], tgt[:, :, 1:2]
    tx2, ty2 = tgt[:, :, 2:3], tgt[:, :, 3:4]
    area_t = (tx2 - tx1) * (ty2 - ty1)

    iw = jnp.minimum(tx2, ax2) - jnp.maximum(tx1, ax1)
    ih = jnp.minimum(ty2, ay2) - jnp.maximum(ty1, ay1)
    inter = jnp.maximum(iw, 0.0) * jnp.maximum(ih, 0.0)
    ov = inter / (area_t + area_a - inter)

    o_iota = jax.lax.broadcasted_iota(jnp.int32, (1, O, 1), 1)
    a_iota = jax.lax.broadcasted_iota(jnp.int32, (1, 1, A), 2)

    BITS2 = 0x40000000
    BITS_HALF = 0x3F000000
    ovb = jax.lax.bitcast_convert_type(ov, jnp.int32)
    key_n = (ovb & ~0xF) | (15 - o_iota)

    m_t = jnp.max(ov, axis=2, keepdims=True)
    bpi = jnp.min(jnp.where(ov == m_t, a_iota, A), axis=2, keepdims=True)
    forced = bpi == a_iota
    key = jnp.where(forced, BITS2 + o_iota, key_n)
    kmax = jnp.max(key, axis=1)

    is_f = kmax >= BITS2
    low = kmax & 0xF
    bti = jnp.where(is_f, low, 15 - low)
    pos = (kmax & ~0xF) >= BITS_HALF

    b0 = (bti & 1) != 0
    b1 = (bti & 2) != 0
    b2 = (bti & 4) != 0
    b3 = (bti & 8) != 0

    def _tree(tc):
        v = [jnp.where(b0, tc[:, 2 * j + 1, :], tc[:, 2 * j, :])
             for j in range(8)]
        v = [jnp.where(b1, v[2 * j + 1], v[2 * j]) for j in range(4)]
        v = [jnp.where(b2, v[2 * j + 1], v[2 * j]) for j in range(2)]
        return jnp.where(b3, v[1], v[0])

    mx1, my1 = _tree(tx1), _tree(ty1)
    mx2, my2 = _tree(tx2), _tree(ty2)

    cx, cy, w, h = cx2, cy2, w2, h2
    g0 = ((mx1 + mx2) * 0.5 - cx) / (_VAR0 * w)
    g1 = ((my1 + my2) * 0.5 - cy) / (_VAR0 * h)
    g2 = jnp.log((mx2 - mx1) / w) / _VAR1
    g3 = jnp.log((my2 - my1) / h) / _VAR1

    loc = loc_ref[:, :, :]
    sl = (_smooth_l1(loc[:, 0, :] - g0) + _smooth_l1(loc[:, 1, :] - g1) +
          _smooth_l1(loc[:, 2, :] - g2) + _smooth_l1(loc[:, 3, :] - g3))
    lsum = jnp.sum(jnp.where(pos, sl, 0.0), axis=1, keepdims=True)
    out_l_ref[:, :] += jnp.sum(lsum, axis=0, keepdims=True)

    c0 = conf_ref[:, 0, :]
    c1 = conf_ref[:, 1, :]
    lse = jnp.maximum(c0, c1) + jnp.log(1.0 + jnp.exp(-jnp.abs(c0 - c1)))
    ce = lse - jnp.where(pos, c1, c0)
    csum = jnp.sum(jnp.where(pos, ce, 0.0), axis=1, keepdims=True)
    out_c_ref[:, :] += jnp.sum(csum, axis=0, keepdims=True)

    ce_ref[pl.ds(step * P, P), :] = jnp.where(pos, 0.0, ce)
    n_pos = jnp.sum(pos.astype(jnp.int32), axis=1, keepdims=True)
    np_ref[pl.ds(step * P, P), :] = jnp.broadcast_to(n_pos, (P, 128))

    @pl.when(step == n_frames // P - 1)
    def _tail():
        npos = np_ref[:, 0:1]
        K = jnp.minimum(npos * _NP_RATIO, A - 1)

        def body(i, t):
            bit = jax.lax.shift_left(jnp.int32(1), jnp.int32(30) - i)
            cand = t + bit
            bits = jax.lax.bitcast_convert_type(ce_ref[:, :], jnp.int32)
            cnt = jnp.sum((bits >= cand).astype(jnp.int32), axis=1,
                          keepdims=True)
            return jnp.where(cnt >= K, cand, t)

        t0 = jnp.zeros((n_frames, 1), jnp.int32)
        t = jax.lax.fori_loop(0, 31, body, t0)
        tf = jax.lax.bitcast_convert_type(t, jnp.float32)
        V = ce_ref[:, :]
        gt = V > tf
        cnt_gt = jnp.sum(jnp.where(gt, 1.0, 0.0), axis=1, keepdims=True)
        sum_gt = jnp.sum(jnp.where(gt, V, 0.0), axis=1, keepdims=True)
        top = sum_gt + (K.astype(jnp.float32) - cnt_gt) * tf
        top = jnp.where(K > 0, top, 0.0)
        out_c_ref[:, :] += jnp.sum(top, axis=0, keepdims=True)


def kernel(loc_data, conf_data, anchors, targets):
    B = targets.shape[0]
    F = targets.shape[1]
    O = targets.shape[2]
    A = anchors.shape[0]
    BF = B * F
    P = _FPB

    loc_p = loc_data.reshape(BF, A, 4).transpose(0, 2, 1)
    conf_p = conf_data.reshape(BF, A, 2).transpose(0, 2, 1)
    tgt = targets.reshape(BF, O, 5)
    anc_t = anchors.T

    out_l, out_c = pl.pallas_call(
        functools.partial(_loss_kernel, n_frames=BF, n_anchors=A, n_objs=O),
        grid=(BF // P,),
        in_specs=[
            pl.BlockSpec((P, O, 5), lambda i: (i, 0, 0)),
            pl.BlockSpec((4, A), lambda i: (0, 0)),
            pl.BlockSpec((P, 4, A), lambda i: (i, 0, 0)),
            pl.BlockSpec((P, 2, A), lambda i: (i, 0, 0)),
        ],
        out_specs=[
            pl.BlockSpec((1, 1), lambda i: (0, 0)),
            pl.BlockSpec((1, 1), lambda i: (0, 0)),
        ],
        out_shape=[
            jax.ShapeDtypeStruct((1, 1), jnp.float32),
            jax.ShapeDtypeStruct((1, 1), jnp.float32),
        ],
        scratch_shapes=[
            pltpu.VMEM((BF, A), jnp.float32),
            pltpu.VMEM((BF, 128), jnp.int32),
        ],
    )(tgt, anc_t, loc_p, conf_p)
    return (out_l[0, 0], out_c[0, 0])
